# Initial kernel scaffold; baseline (speedup 1.0000x reference)
import jax
import jax.numpy as jnp
from jax.experimental import pallas as pl


def kernel(x, edge_index, num_nodes, W, b):
    raise NotImplementedError("write your pallas kernel here")


# speedup vs baseline: 4.7611x; 4.7611x over previous
import functools

import jax
import jax.numpy as jnp
from jax import lax
from jax.experimental import pallas as pl
from jax.experimental.pallas import tpu as pltpu
from jax.experimental.pallas import tpu_sc as plsc

_info = plsc.get_sparse_core_info()
NC = _info.num_cores
NS = _info.num_subcores
NW = NC * NS
B = 128


def _make_sc_agg(n_rows, nb, dp):
    rpt = n_rows // NS

    mesh = plsc.VectorSubcoreMesh(core_axis_name="c", subcore_axis_name="s")

    @functools.partial(
        pl.kernel,
        out_type=jax.ShapeDtypeStruct((NC, n_rows, dp), jnp.float32),
        mesh=mesh,
        scratch_types=[
            pltpu.VMEM((nb, B), jnp.int32),
            pltpu.VMEM((nb, B), jnp.int32),
            pltpu.VMEM((B, dp), jnp.float32),
            pltpu.VMEM_SHARED((n_rows, dp), jnp.float32),
            pltpu.SemaphoreType.DMA,
        ],
        compiler_params=pltpu.CompilerParams(use_tc_tiling_on_sc=False),
    )
    def sc_agg(xp_hbm, srcp_hbm, dstp_hbm, zeros_hbm, out_hbm,
               src_v, dst_v, rows_v, acc_sh, sem):
        c = lax.axis_index("c")
        s = lax.axis_index("s")
        wid = s * NC + c

        pltpu.sync_copy(zeros_hbm.at[pl.ds(s * rpt, rpt)],
                        acc_sh.at[pl.ds(s * rpt, rpt)])
        pltpu.sync_copy(srcp_hbm.at[wid], src_v)
        pltpu.sync_copy(dstp_hbm.at[wid], dst_v)
        plsc.subcore_barrier()

        def body(i, _):
            pltpu.async_copy(xp_hbm.at[src_v.at[i]], rows_v, sem).wait()
            pltpu.sync_copy(rows_v, acc_sh.at[dst_v.at[i]], add=True)
            return 0

        lax.fori_loop(0, nb, body, 0)

        plsc.subcore_barrier()
        pltpu.sync_copy(acc_sh.at[pl.ds(s * rpt, rpt)],
                        out_hbm.at[c, pl.ds(s * rpt, rpt)])

    return sc_agg


def _tc_finish(parts, x, wt, b2, dp, bn):
    n, d = x.shape
    d_out = wt.shape[1]

    def body(p_ref, x_ref, wt_ref, b_ref, o_ref):
        acc = p_ref[0] + p_ref[1]
        deg = jnp.maximum(acc[:, d:d + 1], 1.0)
        agg = acc[:, :d] / deg
        h = jnp.concatenate([agg, x_ref[...]], axis=1)
        o_ref[...] = (
            jnp.dot(h, wt_ref[...], preferred_element_type=jnp.float32)
            + b_ref[...]
        )

    return pl.pallas_call(
        body,
        grid=(n // bn,),
        in_specs=[
            pl.BlockSpec((NC, bn, dp), lambda i: (0, i, 0)),
            pl.BlockSpec((bn, d), lambda i: (i, 0)),
            pl.BlockSpec((2 * d, d_out), lambda i: (0, 0)),
            pl.BlockSpec((1, d_out), lambda i: (0, 0)),
        ],
        out_specs=pl.BlockSpec((bn, d_out), lambda i: (i, 0)),
        out_shape=jax.ShapeDtypeStruct((n, d_out), jnp.float32),
    )(parts, x, wt, b2)


def kernel(x, edge_index, num_nodes, W, b):
    n, d = x.shape
    e = edge_index.shape[1]
    dp = d + 16
    ew = -(-e // NW)
    nb = -(-ew // B)
    e_pad = NW * nb * B
    gran = NS * 8
    n_rows = -(-(n + (1 if e_pad != e else 0)) // gran) * gran

    src = edge_index[0]
    dst = edge_index[1]
    if e_pad != e:
        src = jnp.concatenate([src, jnp.zeros((e_pad - e,), src.dtype)])
        dst = jnp.concatenate([dst, jnp.full((e_pad - e,), n, dst.dtype)])
    srcp = src.reshape(NW, nb, B)
    dstp = dst.reshape(NW, nb, B)

    xp = jnp.concatenate(
        [x, jnp.ones((n, 1), x.dtype), jnp.zeros((n, dp - d - 1), x.dtype)],
        axis=1)
    zeros = jnp.zeros((n_rows, dp), jnp.float32)

    parts = _make_sc_agg(n_rows, nb, dp)(xp, srcp, dstp, zeros)

    wt = W.T
    b2 = b.reshape(1, -1)
    return _tc_finish(parts, x, wt, b2, dp, bn=1000)


# speedup vs baseline: 5.4920x; 1.1535x over previous
import functools

import jax
import jax.numpy as jnp
from jax import lax
from jax.experimental import pallas as pl
from jax.experimental.pallas import tpu as pltpu
from jax.experimental.pallas import tpu_sc as plsc

_info = plsc.get_sparse_core_info()
NC = _info.num_cores
NS = _info.num_subcores
NW = NC * NS
B = 64


def _make_sc_agg(n_rows, nb, dp):
    rpt = n_rows // NS

    mesh = plsc.VectorSubcoreMesh(core_axis_name="c", subcore_axis_name="s")

    @functools.partial(
        pl.kernel,
        out_type=jax.ShapeDtypeStruct((NC, n_rows, dp), jnp.float32),
        mesh=mesh,
        scratch_types=[
            pltpu.VMEM((nb, B), jnp.int32),
            pltpu.VMEM((nb, B), jnp.int32),
            pltpu.VMEM((B, dp), jnp.float32),
            pltpu.VMEM((B, dp), jnp.float32),
            pltpu.VMEM_SHARED((n_rows, dp), jnp.float32),
            pltpu.SemaphoreType.DMA,
            pltpu.SemaphoreType.DMA,
        ],
        compiler_params=pltpu.CompilerParams(use_tc_tiling_on_sc=False),
    )
    def sc_agg(xp_hbm, srcp_hbm, dstp_hbm, zeros_hbm, out_hbm,
               src_v, dst_v, rows0_v, rows1_v, acc_sh, sem0, sem1):
        c = lax.axis_index("c")
        s = lax.axis_index("s")
        wid = s * NC + c

        pltpu.sync_copy(zeros_hbm.at[pl.ds(s * rpt, rpt)],
                        acc_sh.at[pl.ds(s * rpt, rpt)])
        pltpu.sync_copy(srcp_hbm.at[wid], src_v)
        pltpu.sync_copy(dstp_hbm.at[wid], dst_v)
        plsc.subcore_barrier()

        pltpu.async_copy(xp_hbm.at[src_v.at[0]], rows0_v, sem0)

        def body(j, _):
            i = 2 * j
            pltpu.async_copy(xp_hbm.at[src_v.at[i + 1]], rows1_v, sem1)
            pltpu.make_async_copy(xp_hbm.at[src_v.at[i]], rows0_v, sem0).wait()
            pltpu.sync_copy(rows0_v, acc_sh.at[dst_v.at[i]], add=True)

            @pl.when(i + 2 < nb)
            def _():
                pltpu.async_copy(xp_hbm.at[src_v.at[i + 2]], rows0_v, sem0)

            pltpu.make_async_copy(xp_hbm.at[src_v.at[i + 1]], rows1_v,
                                  sem1).wait()
            pltpu.sync_copy(rows1_v, acc_sh.at[dst_v.at[i + 1]], add=True)
            return 0

        lax.fori_loop(0, nb // 2, body, 0)

        plsc.subcore_barrier()
        pltpu.sync_copy(acc_sh.at[pl.ds(s * rpt, rpt)],
                        out_hbm.at[c, pl.ds(s * rpt, rpt)])

    return sc_agg


def _tc_finish(parts, x, wt, b2, dp, bn):
    n, d = x.shape
    d_out = wt.shape[1]

    def body(p_ref, x_ref, wt_ref, b_ref, o_ref):
        acc = p_ref[0] + p_ref[1]
        deg = jnp.maximum(acc[:, d:d + 1], 1.0)
        agg = acc[:, :d] / deg
        h = jnp.concatenate([agg, x_ref[...]], axis=1)
        o_ref[...] = (
            jnp.dot(h, wt_ref[...], preferred_element_type=jnp.float32)
            + b_ref[...]
        )

    return pl.pallas_call(
        body,
        grid=(n // bn,),
        in_specs=[
            pl.BlockSpec((NC, bn, dp), lambda i: (0, i, 0)),
            pl.BlockSpec((bn, d), lambda i: (i, 0)),
            pl.BlockSpec((2 * d, d_out), lambda i: (0, 0)),
            pl.BlockSpec((1, d_out), lambda i: (0, 0)),
        ],
        out_specs=pl.BlockSpec((bn, d_out), lambda i: (i, 0)),
        out_shape=jax.ShapeDtypeStruct((n, d_out), jnp.float32),
    )(parts, x, wt, b2)


def kernel(x, edge_index, num_nodes, W, b):
    n, d = x.shape
    e = edge_index.shape[1]
    dp = d + 16
    ew = -(-e // NW)
    nb = -(-ew // B)
    nb = nb + (nb % 2)
    e_pad = NW * nb * B
    gran = NS * 8
    n_rows = -(-(n + (1 if e_pad != e else 0)) // gran) * gran

    src = edge_index[0]
    dst = edge_index[1]
    if e_pad != e:
        src = jnp.concatenate([src, jnp.zeros((e_pad - e,), src.dtype)])
        dst = jnp.concatenate([dst, jnp.full((e_pad - e,), n, dst.dtype)])
    srcp = src.reshape(NW, nb, B)
    dstp = dst.reshape(NW, nb, B)

    xp = jnp.concatenate(
        [x, jnp.ones((n, 1), x.dtype), jnp.zeros((n, dp - d - 1), x.dtype)],
        axis=1)
    zeros = jnp.zeros((n_rows, dp), jnp.float32)

    parts = _make_sc_agg(n_rows, nb, dp)(xp, srcp, dstp, zeros)

    wt = W.T
    b2 = b.reshape(1, -1)
    return _tc_finish(parts, x, wt, b2, dp, bn=1000)
